# Initial kernel scaffold; baseline (speedup 1.0000x reference)
import functools

import jax
import jax.numpy as jnp
from jax import lax
from jax.experimental import pallas as pl
from jax.experimental.pallas import tpu as pltpu
from jax.experimental.pallas import tpu_sc as plsc

N = 10000
E = 320000
D = 128
K = 3

NC, NS = 2, 16
NW = NC * NS
CH = 125
STEPS = E // (NW * CH)
ROWS_T = N // NS
DEGW = 16

_mesh = plsc.VectorSubcoreMesh(core_axis_name="c", subcore_axis_name="s")


@functools.partial(
    pl.kernel,
    out_type=jax.ShapeDtypeStruct((NC, N, DEGW), jnp.float32),
    mesh=_mesh,
    scratch_types=[
        pltpu.VMEM((STEPS, CH), jnp.int32),
        pltpu.VMEM((CH, DEGW), jnp.float32),
        pltpu.VMEM((CH, DEGW), jnp.float32),
        pltpu.VMEM_SHARED((N, DEGW), jnp.float32),
    ],
)
def _deg_kernel(dst_hbm, out_hbm, dst_v, ones_v, z_v, acc):
    c = lax.axis_index("c")
    s = lax.axis_index("s")
    wid = c * NS + s

    def fill(i, carry):
        ones_v[i, :] = jnp.ones((DEGW,), jnp.float32)
        z_v[i, :] = jnp.zeros((DEGW,), jnp.float32)
        return carry

    lax.fori_loop(0, CH, fill, 0)

    for k in range(ROWS_T // CH):
        pltpu.sync_copy(z_v, acc.at[pl.ds(s * ROWS_T + k * CH, CH)])
    pltpu.sync_copy(dst_hbm.at[pl.ds(wid * STEPS, STEPS)], dst_v)
    plsc.subcore_barrier()

    def step(j, carry):
        pltpu.sync_copy(ones_v, acc.at[dst_v.at[j]], add=True)
        return carry

    lax.fori_loop(0, STEPS, step, 0)
    plsc.subcore_barrier()
    pltpu.sync_copy(acc.at[pl.ds(s * ROWS_T, ROWS_T)],
                    out_hbm.at[c].at[pl.ds(s * ROWS_T, ROWS_T)])


@functools.partial(
    pl.kernel,
    out_type=jax.ShapeDtypeStruct((NC, N, D), jnp.float32),
    mesh=_mesh,
    scratch_types=[
        pltpu.VMEM((STEPS, CH), jnp.int32),
        pltpu.VMEM((STEPS, CH), jnp.int32),
        pltpu.VMEM((CH, D), jnp.float32),
        pltpu.VMEM((CH, D), jnp.float32),
        pltpu.VMEM_SHARED((N, D), jnp.float32),
        pltpu.SemaphoreType.DMA,
    ],
)
def _hop_kernel(src_hbm, dst_hbm, u_hbm, out_hbm,
                src_v, dst_v, rows_v, z_v, acc, sem):
    c = lax.axis_index("c")
    s = lax.axis_index("s")
    wid = c * NS + s

    def fill(i, carry):
        for k in range(D // 16):
            z_v[i, pl.ds(k * 16, 16)] = jnp.zeros((16,), jnp.float32)
        return carry

    lax.fori_loop(0, CH, fill, 0)

    for k in range(ROWS_T // CH):
        pltpu.sync_copy(z_v, acc.at[pl.ds(s * ROWS_T + k * CH, CH)])
    pltpu.sync_copy(src_hbm.at[pl.ds(wid * STEPS, STEPS)], src_v)
    pltpu.sync_copy(dst_hbm.at[pl.ds(wid * STEPS, STEPS)], dst_v)
    plsc.subcore_barrier()

    def step(j, carry):
        pltpu.async_copy(u_hbm.at[src_v.at[j]], rows_v, sem).wait()
        pltpu.sync_copy(rows_v, acc.at[dst_v.at[j]], add=True)
        return carry

    lax.fori_loop(0, STEPS, step, 0)
    plsc.subcore_barrier()
    pltpu.sync_copy(acc.at[pl.ds(s * ROWS_T, ROWS_T)],
                    out_hbm.at[c].at[pl.ds(s * ROWS_T, ROWS_T)])


RB = 1000


def _prep_body(degp_ref, x_ref, u0_ref, dinv_ref, sq_ref):
    deg = degp_ref[0] + degp_ref[1] + 1.0
    di = lax.rsqrt(deg)
    u0_ref[...] = x_ref[...] * di[:, :1]
    dinv_ref[...] = 1.0 / deg
    sq_ref[...] = deg * di


def _prep(degp, x):
    return pl.pallas_call(
        _prep_body,
        grid=(N // RB,),
        in_specs=[
            pl.BlockSpec((NC, RB, DEGW), lambda i: (0, i, 0)),
            pl.BlockSpec((RB, D), lambda i: (i, 0)),
        ],
        out_specs=[
            pl.BlockSpec((RB, D), lambda i: (i, 0)),
            pl.BlockSpec((RB, DEGW), lambda i: (i, 0)),
            pl.BlockSpec((RB, DEGW), lambda i: (i, 0)),
        ],
        out_shape=[
            jax.ShapeDtypeStruct((N, D), jnp.float32),
            jax.ShapeDtypeStruct((N, DEGW), jnp.float32),
            jax.ShapeDtypeStruct((N, DEGW), jnp.float32),
        ],
    )(degp, x)


def _combine_body(p_ref, u_ref, dinv_ref, out_ref):
    out_ref[...] = (p_ref[0] + p_ref[1] + u_ref[...]) * dinv_ref[:, :1]


def _combine(p, u, dinv):
    return pl.pallas_call(
        _combine_body,
        grid=(N // RB,),
        in_specs=[
            pl.BlockSpec((NC, RB, D), lambda i: (0, i, 0)),
            pl.BlockSpec((RB, D), lambda i: (i, 0)),
            pl.BlockSpec((RB, DEGW), lambda i: (i, 0)),
        ],
        out_specs=pl.BlockSpec((RB, D), lambda i: (i, 0)),
        out_shape=jax.ShapeDtypeStruct((N, D), jnp.float32),
    )(p, u, dinv)


def _final_body(u_ref, sq_ref, w_ref, b_ref, out_ref):
    h = u_ref[...] * sq_ref[:, :1]
    out_ref[...] = (
        jnp.dot(h, w_ref[...], preferred_element_type=jnp.float32) + b_ref[...]
    )


def _final(u, sq, W, b2):
    return pl.pallas_call(
        _final_body,
        grid=(N // RB,),
        in_specs=[
            pl.BlockSpec((RB, D), lambda i: (i, 0)),
            pl.BlockSpec((RB, DEGW), lambda i: (i, 0)),
            pl.BlockSpec((D, D), lambda i: (0, 0)),
            pl.BlockSpec((1, D), lambda i: (0, 0)),
        ],
        out_specs=pl.BlockSpec((RB, D), lambda i: (i, 0)),
        out_shape=jax.ShapeDtypeStruct((N, D), jnp.float32),
    )(u, sq, W, b2)


def kernel(x, edge_index, W, b):
    src = edge_index[0].reshape(NW * STEPS, CH)
    dst = edge_index[1].reshape(NW * STEPS, CH)
    degp = _deg_kernel(dst)
    u, dinv, sq = _prep(degp, x)
    for _ in range(K):
        p = _hop_kernel(src, dst, u)
        u = _combine(p, u, dinv)
    return _final(u, sq, W, b.reshape(1, D))


# speedup vs baseline: 8.0323x; 8.0323x over previous
import functools

import jax
import jax.numpy as jnp
from jax import lax
from jax.experimental import pallas as pl
from jax.experimental.pallas import tpu as pltpu
from jax.experimental.pallas import tpu_sc as plsc

N = 10000
E = 320000
D = 128
K = 3

NC, NS = 2, 16
NW = NC * NS
CH = 128
STEPS = 80
EPAD = NW * STEPS * CH
NPAD = 10112
ROWS_T = NPAD // NS
DEGW = 16

_sc_cache = {}


def _sc_kernels():
    if "k" in _sc_cache:
        return _sc_cache["k"]

    mesh = plsc.VectorSubcoreMesh(
        core_axis_name="c", subcore_axis_name="s",
        num_cores=NC, num_subcores=NS)

    @functools.partial(
        pl.kernel,
        out_type=jax.ShapeDtypeStruct((NC, NPAD, DEGW), jnp.float32),
        mesh=mesh,
        scratch_types=[
            pltpu.VMEM((STEPS, CH), jnp.int32),
            pltpu.VMEM((CH, DEGW), jnp.float32),
            pltpu.VMEM_SHARED((NPAD, DEGW), jnp.float32),
        ],
    )
    def deg_kernel(dst_hbm, out_hbm, dst_v, ones_v, acc):
        c = lax.axis_index("c")
        s = lax.axis_index("s")
        wid = c * NS + s

        def fillz(i, carry):
            ones_v[i, :] = jnp.zeros((DEGW,), jnp.float32)
            return carry

        lax.fori_loop(0, CH, fillz, 0)
        for k in range(ROWS_T // CH):
            pltpu.sync_copy(ones_v, acc.at[pl.ds(s * ROWS_T + k * CH, CH)])
        rem = ROWS_T % CH
        if rem:
            pltpu.sync_copy(
                ones_v.at[pl.ds(0, rem)],
                acc.at[pl.ds(s * ROWS_T + (ROWS_T // CH) * CH, rem)])

        def fill1(i, carry):
            ones_v[i, :] = jnp.ones((DEGW,), jnp.float32)
            return carry

        lax.fori_loop(0, CH, fill1, 0)
        pltpu.sync_copy(dst_hbm.at[pl.ds(wid * STEPS, STEPS)], dst_v)
        plsc.subcore_barrier()

        def step(j, carry):
            pltpu.sync_copy(ones_v, acc.at[dst_v.at[j]], add=True)
            return carry

        lax.fori_loop(0, STEPS, step, 0)
        plsc.subcore_barrier()
        pltpu.sync_copy(acc.at[pl.ds(s * ROWS_T, ROWS_T)],
                        out_hbm.at[c].at[pl.ds(s * ROWS_T, ROWS_T)])

    GS = 8
    NG = STEPS // GS

    @functools.partial(
        pl.kernel,
        out_type=jax.ShapeDtypeStruct((NC, NPAD, D), jnp.float32),
        mesh=mesh,
        scratch_types=[
            pltpu.VMEM((2, GS, CH), jnp.int32),
            pltpu.VMEM((2, GS, CH), jnp.int32),
            pltpu.VMEM((CH, D), jnp.float32),
            pltpu.VMEM((CH, D), jnp.float32),
            pltpu.VMEM_SHARED((NPAD, D), jnp.float32),
            pltpu.SemaphoreType.DMA,
            pltpu.SemaphoreType.DMA,
            pltpu.SemaphoreType.DMA,
            pltpu.SemaphoreType.DMA,
        ],
    )
    def hop_kernel(src_hbm, dst_hbm, u_hbm, out_hbm,
                   sidx, didx, rows0, rows1, acc, isem0, isem1, gsem0, gsem1):
        c = lax.axis_index("c")
        s = lax.axis_index("s")
        wid = c * NS + s
        base = wid * STEPS
        rows = (rows0, rows1)
        gsems = (gsem0, gsem1)
        isems = (isem0, isem1)

        def fillz(i, carry):
            for k in range(D // 16):
                rows0[i, pl.ds(k * 16, 16)] = jnp.zeros((16,), jnp.float32)
            return carry

        lax.fori_loop(0, CH, fillz, 0)
        for k in range(ROWS_T // CH):
            pltpu.sync_copy(rows0, acc.at[pl.ds(s * ROWS_T + k * CH, CH)])
        rem = ROWS_T % CH
        if rem:
            pltpu.sync_copy(
                rows0.at[pl.ds(0, rem)],
                acc.at[pl.ds(s * ROWS_T + (ROWS_T // CH) * CH, rem)])

        def pfg(g, slot):
            off = pl.multiple_of(base + g * GS, GS)
            pltpu.async_copy(src_hbm.at[pl.ds(off, GS)], sidx.at[slot],
                             isems[slot])
            pltpu.async_copy(dst_hbm.at[pl.ds(off, GS)], didx.at[slot],
                             isems[slot])

        def pfg_wait(g, slot):
            off = pl.multiple_of(base + g * GS, GS)
            pltpu.make_async_copy(src_hbm.at[pl.ds(off, GS)], sidx.at[slot],
                                  isems[slot]).wait()
            pltpu.make_async_copy(src_hbm.at[pl.ds(off, GS)], didx.at[slot],
                                  isems[slot]).wait()

        def g_start(slot, row, par):
            pltpu.async_copy(u_hbm.at[sidx.at[slot, row]], rows[par],
                             gsems[par])

        def g_wait(slot, row, par):
            pltpu.make_async_copy(u_hbm.at[sidx.at[slot, row]], rows[par],
                                  gsems[par]).wait()

        def scat(slot, row, par):
            pltpu.sync_copy(rows[par], acc.at[didx.at[slot, row]], add=True)

        def group(g, slot, refill, wait_next, last_start):
            for p in range(GS):
                if p == GS - 2 and wait_next:
                    pfg_wait(g + 1, slot ^ 1)
                if p < GS - 1:
                    g_start(slot, p + 1, (p + 1) % 2)
                elif last_start:
                    g_start(slot ^ 1, 0, 0)
                g_wait(slot, p, p % 2)
                scat(slot, p, p % 2)
            if refill:
                pfg(g + 2, slot)

        plsc.subcore_barrier()

        pfg(0, 0)
        pfg(1, 1)
        pfg_wait(0, 0)
        g_start(0, 0, 0)

        def super_group(i, carry):
            g = 2 * i
            group(g, 0, True, True, True)
            group(g + 1, 1, True, True, True)
            return carry

        lax.fori_loop(0, NG // 2 - 1, super_group, 0)
        group(NG - 2, 0, False, True, True)
        group(NG - 1, 1, False, False, False)

        plsc.subcore_barrier()
        pltpu.sync_copy(acc.at[pl.ds(s * ROWS_T, ROWS_T)],
                        out_hbm.at[c].at[pl.ds(s * ROWS_T, ROWS_T)])

    _sc_cache["k"] = (deg_kernel, hop_kernel)
    return _sc_cache["k"]


RB = 1000


def _prep_body(degp_ref, x_ref, u0_ref, dinv_ref, sq_ref):
    deg = degp_ref[0] + degp_ref[1] + 1.0
    di = lax.rsqrt(deg)
    u0_ref[...] = x_ref[...] * di[:, :1]
    dinv_ref[...] = 1.0 / deg
    sq_ref[...] = deg * di


def _prep(degp, x):
    return pl.pallas_call(
        _prep_body,
        grid=(N // RB,),
        in_specs=[
            pl.BlockSpec((NC, RB, DEGW), lambda i: (0, i, 0)),
            pl.BlockSpec((RB, D), lambda i: (i, 0)),
        ],
        out_specs=[
            pl.BlockSpec((RB, D), lambda i: (i, 0)),
            pl.BlockSpec((RB, DEGW), lambda i: (i, 0)),
            pl.BlockSpec((RB, DEGW), lambda i: (i, 0)),
        ],
        out_shape=[
            jax.ShapeDtypeStruct((N, D), jnp.float32),
            jax.ShapeDtypeStruct((N, DEGW), jnp.float32),
            jax.ShapeDtypeStruct((N, DEGW), jnp.float32),
        ],
    )(degp, x)


def _combine_body(p_ref, u_ref, dinv_ref, out_ref):
    out_ref[...] = (p_ref[0] + p_ref[1] + u_ref[...]) * dinv_ref[:, :1]


def _combine(p, u, dinv):
    return pl.pallas_call(
        _combine_body,
        grid=(N // RB,),
        in_specs=[
            pl.BlockSpec((NC, RB, D), lambda i: (0, i, 0)),
            pl.BlockSpec((RB, D), lambda i: (i, 0)),
            pl.BlockSpec((RB, DEGW), lambda i: (i, 0)),
        ],
        out_specs=pl.BlockSpec((RB, D), lambda i: (i, 0)),
        out_shape=jax.ShapeDtypeStruct((N, D), jnp.float32),
    )(p, u, dinv)


def _final_body(u_ref, sq_ref, w_ref, b_ref, out_ref):
    h = u_ref[...] * sq_ref[:, :1]
    out_ref[...] = (
        jnp.dot(h, w_ref[...], preferred_element_type=jnp.float32) + b_ref[...]
    )


def _final(u, sq, W, b2):
    return pl.pallas_call(
        _final_body,
        grid=(N // RB,),
        in_specs=[
            pl.BlockSpec((RB, D), lambda i: (i, 0)),
            pl.BlockSpec((RB, DEGW), lambda i: (i, 0)),
            pl.BlockSpec((D, D), lambda i: (0, 0)),
            pl.BlockSpec((1, D), lambda i: (0, 0)),
        ],
        out_specs=pl.BlockSpec((RB, D), lambda i: (i, 0)),
        out_shape=jax.ShapeDtypeStruct((N, D), jnp.float32),
    )(u, sq, W, b2)


def kernel(x, edge_index, W, b):
    npad_e = EPAD - E
    src = jnp.concatenate(
        [edge_index[0], jnp.zeros((npad_e,), jnp.int32)]).reshape(
            NW * STEPS, CH)
    dst = jnp.concatenate(
        [edge_index[1],
         N + (jnp.arange(npad_e, dtype=jnp.int32) % (NPAD - N))]).reshape(
             NW * STEPS, CH)
    deg_kernel, hop_kernel = _sc_kernels()
    degp = deg_kernel(dst)
    u, dinv, sq = _prep(degp, x)
    for _ in range(K):
        p = hop_kernel(src, dst, u)
        u = _combine(p, u, dinv)
    return _final(u, sq, W, b.reshape(1, D))
